# Initial kernel scaffold; baseline (speedup 1.0000x reference)
import jax
import jax.numpy as jnp
from jax.experimental import pallas as pl


def kernel(one_hot, features, gemme_features, a_res, W_enc0, b_enc0, W_enc1, b_enc1, W_enc2, b_enc2, W_enc3, b_enc3, W_g0, b_g0, W_g1, b_g1, W_g2, b_g2, W_g3, b_g3, W_g4, b_g4, W_g5, b_g5, W_g6, b_g6, W_g7, b_g7, W_fin0, b_fin0, W_fin1, b_fin1, W_fin2, b_fin2, W_fin3, b_fin3, W_fin4, b_fin4):
    raise NotImplementedError("write your pallas kernel here")


# speedup vs baseline: 4.6486x; 4.6486x over previous
import functools

import jax
import jax.numpy as jnp
from jax import lax
from jax.experimental import pallas as pl
from jax.experimental.pallas import tpu as pltpu
from jax.experimental.pallas import tpu_sc as plsc

N = 50000
E = 800000

CHUNK = 4096
NUM_CHUNKS = 13
N_PAD = CHUNK * NUM_CHUNKS
T = 128
EPAD_EXTRA = 16 * T
E_PAD = E + EPAD_EXTRA
NS = 16
OB = 64
ROWS_PER_W = CHUNK // NS
NK = ROWS_PER_W // OB

_IOTA16 = None


def _pad2(w, r, c):
    return jnp.pad(w, ((0, r - w.shape[0]), (0, c - w.shape[1])))


def _pad1(b, c):
    return jnp.pad(b, (0, c - b.shape[0]))


_R = 512


def _enc_body(x_ref, w0, b0, w1, b1, w2, b2, w3, b3, o_ref):
    x = x_ref[...]
    for w, b in ((w0, b0), (w1, b1), (w2, b2), (w3, b3)):
        x = jnp.maximum(
            jnp.dot(x, w[...], preferred_element_type=jnp.float32) + b[...], 0.0)
    o_ref[...] = x


def _encoder(x0, ws):
    din = x0.shape[1]
    specs = [pl.BlockSpec((_R, din), lambda i: (i, 0))]
    for w, b in ws:
        specs.append(pl.BlockSpec(w.shape, lambda i: (0, 0)))
        specs.append(pl.BlockSpec(b.shape, lambda i: (0, 0)))
    dout = ws[-1][0].shape[1]
    args = []
    for w, b in ws:
        args += [w, b]
    return pl.pallas_call(
        _enc_body,
        grid=(N_PAD // _R,),
        in_specs=specs,
        out_specs=pl.BlockSpec((_R, dout), lambda i: (i, 0)),
        out_shape=jax.ShapeDtypeStruct((N_PAD, dout), jnp.float32),
    )(x0, *args)


def _mm_body(x_ref, w_ref, b_ref, o_ref):
    x = jnp.maximum(x_ref[...], 0.0)
    o_ref[...] = (
        jnp.dot(x, w_ref[...], preferred_element_type=jnp.float32) + b_ref[...])


def _relu_mm(x, w, b):
    din, dout = w.shape
    return pl.pallas_call(
        _mm_body,
        grid=(N_PAD // _R,),
        in_specs=[
            pl.BlockSpec((_R, din), lambda i: (i, 0)),
            pl.BlockSpec((din, dout), lambda i: (0, 0)),
            pl.BlockSpec((1, dout), lambda i: (0, 0)),
        ],
        out_specs=pl.BlockSpec((_R, dout), lambda i: (i, 0)),
        out_shape=jax.ShapeDtypeStruct((N_PAD, dout), jnp.float32),
    )(x, w, b)


def _fin_body(x_ref, w0, b0, w1, b1, w2, b2, w3, b3, w4, b4, o_ref):
    x = jnp.maximum(x_ref[...], 0.0)
    for w, b in ((w0, b0), (w1, b1), (w2, b2), (w3, b3)):
        x = jnp.maximum(
            jnp.dot(x, w[...], preferred_element_type=jnp.float32) + b[...], 0.0)
    x = jnp.dot(x, w4[...], preferred_element_type=jnp.float32) + b4[...]
    o_ref[...] = jax.nn.sigmoid(x)


def _final(x, ws):
    din = x.shape[1]
    specs = [pl.BlockSpec((_R, din), lambda i: (i, 0))]
    args = []
    for w, b in ws:
        specs.append(pl.BlockSpec(w.shape, lambda i: (0, 0)))
        specs.append(pl.BlockSpec(b.shape, lambda i: (0, 0)))
        args += [w, b]
    dout = ws[-1][0].shape[1]
    return pl.pallas_call(
        _fin_body,
        grid=(N_PAD // _R,),
        in_specs=specs,
        out_specs=pl.BlockSpec((_R, dout), lambda i: (i, 0)),
        out_shape=jax.ShapeDtypeStruct((N_PAD, dout), jnp.float32),
    )(x, *args)


def _extract(vec, k):
    return jax.lax.squeeze(jax.lax.slice(vec, (k,), (k + 1,)), (0,))


def _make_segsum(d):
    mesh = plsc.VectorSubcoreMesh(core_axis_name="c", subcore_axis_name="s",
                                  num_cores=2)

    def body(h_hbm, src_hbm, dst_hbm, cptr_hbm, out_hbm,
             acc, src_v, dst_v, ldst_v, rows_v, zbuf, obuf, cptr_v, sem):
        cid = lax.axis_index("c")
        wid = lax.axis_index("s")
        iota = lax.iota(jnp.int32, 16)

        def zrow(r, carry):
            for j in range(d // 16):
                zbuf[r, pl.ds(j * 16, 16)] = jnp.zeros((16,), jnp.float32)
            return carry
        lax.fori_loop(0, OB, zrow, 0)

        pltpu.sync_copy(cptr_hbm, cptr_v)
        cvec = cptr_v[...]

        for c in range(NUM_CHUNKS):
            @pl.when(cid == c % 2)
            def _chunk(c=c):
                base = c * CHUNK
                lo = _extract(cvec, c)
                hi = _extract(cvec, c + 1)
                for k in range(NK):
                    pltpu.sync_copy(
                        zbuf, acc.at[pl.ds(wid * ROWS_PER_W + k * OB, OB)])
                plsc.subcore_barrier()

                lo8 = (lo >> 3) << 3
                nt = (hi - lo8 + (NS * T - 1)) >> 11

                def tile(t, carry):
                    s = pl.multiple_of(lo8 + (t * NS + wid) * T, 8)
                    pltpu.sync_copy(src_hbm.at[pl.ds(s, T)], src_v)
                    pltpu.sync_copy(dst_hbm.at[pl.ds(s, T)], dst_v)
                    for j in range(T // 16):
                        v = dst_v[pl.ds(j * 16, 16)]
                        inb = (v >= base) & (v < base + CHUNK)
                        ldst_v[pl.ds(j * 16, 16)] = jnp.where(
                            inb, v - base, CHUNK + iota)
                    pltpu.async_copy(h_hbm.at[src_v], rows_v, sem).wait()
                    pltpu.sync_copy(rows_v, acc.at[ldst_v], add=True)
                    return carry

                lax.fori_loop(0, nt, tile, 0)
                plsc.subcore_barrier()

                for k in range(NK):
                    r0 = wid * ROWS_PER_W + k * OB
                    pltpu.sync_copy(acc.at[pl.ds(r0, OB)], obuf)
                    pltpu.sync_copy(obuf, out_hbm.at[pl.ds(base + r0, OB)])
                plsc.subcore_barrier()

    return pl.kernel(
        body,
        out_type=jax.ShapeDtypeStruct((N_PAD, d), jnp.float32),
        mesh=mesh,
        compiler_params=pltpu.CompilerParams(use_tc_tiling_on_sc=False),
        scratch_types=[
            pltpu.VMEM_SHARED((CHUNK + 16, d), jnp.float32),
            pltpu.VMEM((T,), jnp.int32),
            pltpu.VMEM((T,), jnp.int32),
            pltpu.VMEM((T,), jnp.int32),
            pltpu.VMEM((T, d), jnp.float32),
            pltpu.VMEM((OB, d), jnp.float32),
            pltpu.VMEM((OB, d), jnp.float32),
            pltpu.VMEM((16,), jnp.int32),
            pltpu.SemaphoreType.DMA,
        ],
    )


_CONV_OUT_PAD = [208, 160, 128, 112, 64, 32, 16, 16]


def kernel(one_hot, features, gemme_features, a_res,
           W_enc0, b_enc0, W_enc1, b_enc1, W_enc2, b_enc2, W_enc3, b_enc3,
           W_g0, b_g0, W_g1, b_g1, W_g2, b_g2, W_g3, b_g3,
           W_g4, b_g4, W_g5, b_g5, W_g6, b_g6, W_g7, b_g7,
           W_fin0, b_fin0, W_fin1, b_fin1, W_fin2, b_fin2, W_fin3, b_fin3,
           W_fin4, b_fin4):
    f32 = jnp.float32

    dst_s, src_s = lax.sort((a_res[1], a_res[0]), num_keys=1)
    bounds = jnp.arange(1, NUM_CHUNKS, dtype=jnp.int32) * CHUNK
    cs = jnp.searchsorted(dst_s, bounds).astype(jnp.int32)
    cptr = jnp.concatenate([
        jnp.zeros((1,), jnp.int32), cs,
        jnp.full((16 - NUM_CHUNKS,), E, jnp.int32)])
    pad_src = (jnp.arange(EPAD_EXTRA, dtype=jnp.int32) * 977) % N
    pad_dst = jnp.full((EPAD_EXTRA,), N_PAD, jnp.int32)
    src_p = jnp.concatenate([src_s, pad_src])
    dst_p = jnp.concatenate([dst_s, pad_dst])

    x0 = jnp.zeros((N_PAD, 160), f32)
    x0 = x0.at[:N, :20].set(one_hot).at[:N, 20:148].set(features)
    enc_ws = [
        (_pad2(W_enc0, 160, 32), _pad1(b_enc0, 32).reshape(1, -1)),
        (W_enc1, b_enc1.reshape(1, -1)),
        (W_enc2, b_enc2.reshape(1, -1)),
        (W_enc3, b_enc3.reshape(1, -1)),
    ]
    x = _encoder(x0, enc_ws)

    gw = [(W_g0, b_g0), (W_g1, b_g1), (W_g2, b_g2), (W_g3, b_g3),
          (W_g4, b_g4), (W_g5, b_g5), (W_g6, b_g6), (W_g7, b_g7)]
    din_pad = 256
    for i, (w, b) in enumerate(gw):
        dout = _CONV_OUT_PAD[i]
        wp = _pad2(w, din_pad, dout)
        bp = _pad1(b, dout).reshape(1, -1)
        h = _relu_mm(x, wp, bp)
        x = _make_segsum(dout)(h, src_p, dst_p, cptr)
        din_pad = dout

    fin_ws = [
        (_pad2(W_fin0, 16, 16), b_fin0.reshape(1, -1)),
        (W_fin1, b_fin1.reshape(1, -1)),
        (W_fin2, b_fin2.reshape(1, -1)),
        (W_fin3, b_fin3.reshape(1, -1)),
        (_pad2(W_fin4, 8, 8), _pad1(b_fin4, 8).reshape(1, -1)),
    ]
    out = _final(x, fin_ws)
    return out[:N, :1]


# speedup vs baseline: 6.4124x; 1.3794x over previous
import functools

import jax
import jax.numpy as jnp
from jax import lax
from jax.experimental import pallas as pl
from jax.experimental.pallas import tpu as pltpu
from jax.experimental.pallas import tpu_sc as plsc

N = 50000
E = 800000

T = 128
IB = 8
IBT = IB * T
NS = 16
EPAD_EXTRA = 3 * NS * IBT
E_PAD = E + EPAD_EXTRA
OB = 32
DSTPAD = 65536

_CHUNK_CFG = {208: (4096, 13), 160: (7168, 7), 128: (10240, 5),
              112: (12544, 4), 64: (25600, 2), 32: (25088, 2),
              16: (25088, 2)}


def _pad2(w, r, c):
    return jnp.pad(w, ((0, r - w.shape[0]), (0, c - w.shape[1])))


def _pad1(b, c):
    return jnp.pad(b, (0, c - b.shape[0]))


_R = 512


def _enc_body(x_ref, w0, b0, w1, b1, w2, b2, w3, b3, o_ref):
    x = x_ref[...]
    for w, b in ((w0, b0), (w1, b1), (w2, b2), (w3, b3)):
        x = jnp.maximum(
            jnp.dot(x, w[...], preferred_element_type=jnp.float32) + b[...], 0.0)
    o_ref[...] = x


def _encoder(x0, ws):
    nrows, din = x0.shape
    specs = [pl.BlockSpec((_R, din), lambda i: (i, 0))]
    for w, b in ws:
        specs.append(pl.BlockSpec(w.shape, lambda i: (0, 0)))
        specs.append(pl.BlockSpec(b.shape, lambda i: (0, 0)))
    dout = ws[-1][0].shape[1]
    args = []
    for w, b in ws:
        args += [w, b]
    return pl.pallas_call(
        _enc_body,
        grid=(nrows // _R,),
        in_specs=specs,
        out_specs=pl.BlockSpec((_R, dout), lambda i: (i, 0)),
        out_shape=jax.ShapeDtypeStruct((nrows, dout), jnp.float32),
    )(x0, *args)


def _mm_body(x_ref, w_ref, b_ref, o_ref):
    x = jnp.maximum(x_ref[...], 0.0)
    o_ref[...] = (
        jnp.dot(x, w_ref[...], preferred_element_type=jnp.float32) + b_ref[...])


def _relu_mm(x, w, b):
    din, dout = w.shape
    nrows = x.shape[0]
    return pl.pallas_call(
        _mm_body,
        grid=(nrows // _R,),
        in_specs=[
            pl.BlockSpec((_R, din), lambda i: (i, 0)),
            pl.BlockSpec((din, dout), lambda i: (0, 0)),
            pl.BlockSpec((1, dout), lambda i: (0, 0)),
        ],
        out_specs=pl.BlockSpec((_R, dout), lambda i: (i, 0)),
        out_shape=jax.ShapeDtypeStruct((nrows, dout), jnp.float32),
    )(x, w, b)


def _fin_body(x_ref, w0, b0, w1, b1, w2, b2, w3, b3, w4, b4, o_ref):
    x = jnp.maximum(x_ref[...], 0.0)
    for w, b in ((w0, b0), (w1, b1), (w2, b2), (w3, b3)):
        x = jnp.maximum(
            jnp.dot(x, w[...], preferred_element_type=jnp.float32) + b[...], 0.0)
    x = jnp.dot(x, w4[...], preferred_element_type=jnp.float32) + b4[...]
    o_ref[...] = jax.nn.sigmoid(x)


def _final(x, ws):
    nrows, din = x.shape
    specs = [pl.BlockSpec((_R, din), lambda i: (i, 0))]
    args = []
    for w, b in ws:
        specs.append(pl.BlockSpec(w.shape, lambda i: (0, 0)))
        specs.append(pl.BlockSpec(b.shape, lambda i: (0, 0)))
        args += [w, b]
    dout = ws[-1][0].shape[1]
    return pl.pallas_call(
        _fin_body,
        grid=(nrows // _R,),
        in_specs=specs,
        out_specs=pl.BlockSpec((_R, dout), lambda i: (i, 0)),
        out_shape=jax.ShapeDtypeStruct((nrows, dout), jnp.float32),
    )(x, *args)


def _extract(vec, k):
    return jax.lax.squeeze(jax.lax.slice(vec, (k,), (k + 1,)), (0,))


def _make_segsum(d):
    chunk, num_chunks = _CHUNK_CFG[d]
    n_pad = chunk * num_chunks
    rows_per_w = chunk // NS
    nk = rows_per_w // OB
    mesh = plsc.VectorSubcoreMesh(core_axis_name="c", subcore_axis_name="s",
                                  num_cores=2)

    def body(h_hbm, src_hbm, dst_hbm, cptr_hbm, out_hbm,
             acc, srcb, dstb, ldstb, rb, zbuf, cptr_v,
             isem0, isem1, gsem0, gsem1):
        cid = lax.axis_index("c")
        wid = lax.axis_index("s")
        iota = lax.iota(jnp.int32, 16)
        isem = (isem0, isem1)
        gsem = (gsem0, gsem1)

        def zrow(r, carry):
            for j in range(d // 16):
                zbuf[r, pl.ds(j * 16, 16)] = jnp.zeros((16,), jnp.float32)
            return carry
        lax.fori_loop(0, OB, zrow, 0)

        pltpu.sync_copy(cptr_hbm, cptr_v)
        cvec = cptr_v[...]

        def ldst_block(p, base):
            for j in range(IB):
                for q in range(T // 16):
                    v = dstb[p][pl.ds(j * T + q * 16, 16)]
                    inb = (v >= base) & (v < base + chunk)
                    ldstb[p][j, pl.ds(q * 16, 16)] = jnp.where(
                        inb, v - base, chunk + iota)

        for c in range(num_chunks):
            @pl.when(cid == c % 2)
            def _chunk(c=c):
                base = c * chunk
                lo = _extract(cvec, c)
                hi = _extract(cvec, c + 1)
                for k in range(nk):
                    pltpu.sync_copy(
                        zbuf, acc.at[pl.ds(wid * rows_per_w + k * OB, OB)])
                plsc.subcore_barrier()

                lo8 = (lo >> 3) << 3

                def blk_start(n):
                    return pl.multiple_of(lo8 + (n * NS + wid) * IBT, 8)

                nb = (hi - lo8 + (NS * IBT - 1)) >> 14
                nb2 = (nb + 1) >> 1

                s0 = blk_start(0)
                pltpu.async_copy(src_hbm.at[pl.ds(s0, IBT)], srcb[0], isem[0])
                pltpu.async_copy(dst_hbm.at[pl.ds(s0, IBT)], dstb[0], isem[0])

                def block_pair(n2, carry):
                    for p in range(2):
                        n = 2 * n2 + p
                        sn = blk_start(n)
                        pltpu.make_async_copy(
                            src_hbm.at[pl.ds(sn, IBT)], srcb[p],
                            isem[p]).wait()
                        pltpu.make_async_copy(
                            dst_hbm.at[pl.ds(sn, IBT)], dstb[p],
                            isem[p]).wait()
                        sn1 = blk_start(n + 1)
                        pltpu.async_copy(
                            src_hbm.at[pl.ds(sn1, IBT)], srcb[p ^ 1],
                            isem[p ^ 1])
                        pltpu.async_copy(
                            dst_hbm.at[pl.ds(sn1, IBT)], dstb[p ^ 1],
                            isem[p ^ 1])
                        ldst_block(p, base)
                        prev = None
                        for j in range(IB):
                            cp = pltpu.async_copy(
                                h_hbm.at[srcb[p].at[pl.ds(j * T, T)]],
                                rb[j % 2], gsem[j % 2])
                            if prev is not None:
                                prev[0].wait()
                                pltpu.sync_copy(
                                    rb[(j - 1) % 2],
                                    acc.at[ldstb[p].at[j - 1]], add=True)
                            prev = (cp, j)
                        prev[0].wait()
                        pltpu.sync_copy(
                            rb[(IB - 1) % 2],
                            acc.at[ldstb[p].at[IB - 1]], add=True)
                    return carry

                lax.fori_loop(0, nb2, block_pair, 0)
                se = blk_start(2 * nb2)
                pltpu.make_async_copy(
                    src_hbm.at[pl.ds(se, IBT)], srcb[0], isem[0]).wait()
                pltpu.make_async_copy(
                    dst_hbm.at[pl.ds(se, IBT)], dstb[0], isem[0]).wait()
                plsc.subcore_barrier()

                for k in range(nk):
                    r0 = wid * rows_per_w + k * OB
                    pltpu.sync_copy(acc.at[pl.ds(r0, OB)],
                                    out_hbm.at[pl.ds(base + r0, OB)])
                plsc.subcore_barrier()

    return pl.kernel(
        body,
        out_type=jax.ShapeDtypeStruct((n_pad, d), jnp.float32),
        mesh=mesh,
        compiler_params=pltpu.CompilerParams(use_tc_tiling_on_sc=False),
        scratch_types=[
            pltpu.VMEM_SHARED((chunk + 16, d), jnp.float32),
            (pltpu.VMEM((IBT,), jnp.int32), pltpu.VMEM((IBT,), jnp.int32)),
            (pltpu.VMEM((IBT,), jnp.int32), pltpu.VMEM((IBT,), jnp.int32)),
            (pltpu.VMEM((IB, T), jnp.int32), pltpu.VMEM((IB, T), jnp.int32)),
            (pltpu.VMEM((T, d), jnp.float32), pltpu.VMEM((T, d), jnp.float32)),
            pltpu.VMEM((OB, d), jnp.float32),
            pltpu.VMEM((16,), jnp.int32),
            pltpu.SemaphoreType.DMA,
            pltpu.SemaphoreType.DMA,
            pltpu.SemaphoreType.DMA,
            pltpu.SemaphoreType.DMA,
        ],
    )


_CONV_OUT_PAD = [208, 160, 128, 112, 64, 32, 16, 16]


def kernel(one_hot, features, gemme_features, a_res,
           W_enc0, b_enc0, W_enc1, b_enc1, W_enc2, b_enc2, W_enc3, b_enc3,
           W_g0, b_g0, W_g1, b_g1, W_g2, b_g2, W_g3, b_g3,
           W_g4, b_g4, W_g5, b_g5, W_g6, b_g6, W_g7, b_g7,
           W_fin0, b_fin0, W_fin1, b_fin1, W_fin2, b_fin2, W_fin3, b_fin3,
           W_fin4, b_fin4):
    f32 = jnp.float32

    dst_s, src_s = lax.sort((a_res[1], a_res[0]), num_keys=1)
    cptrs = {}
    for chunk, num_chunks in set(_CHUNK_CFG.values()):
        bounds = jnp.arange(1, num_chunks, dtype=jnp.int32) * chunk
        cs = jnp.searchsorted(dst_s, bounds).astype(jnp.int32)
        cptrs[chunk] = jnp.concatenate([
            jnp.zeros((1,), jnp.int32), cs,
            jnp.full((16 - num_chunks,), E, jnp.int32)])
    pad_src = (jnp.arange(EPAD_EXTRA, dtype=jnp.int32) * 977) % N
    pad_dst = jnp.full((EPAD_EXTRA,), DSTPAD, jnp.int32)
    src_p = jnp.concatenate([src_s, pad_src])
    dst_p = jnp.concatenate([dst_s, pad_dst])

    x0 = jnp.zeros((50176, 160), f32)
    x0 = x0.at[:N, :20].set(one_hot).at[:N, 20:148].set(features)
    enc_ws = [
        (_pad2(W_enc0, 160, 32), _pad1(b_enc0, 32).reshape(1, -1)),
        (W_enc1, b_enc1.reshape(1, -1)),
        (W_enc2, b_enc2.reshape(1, -1)),
        (W_enc3, b_enc3.reshape(1, -1)),
    ]
    x = _encoder(x0, enc_ws)

    gw = [(W_g0, b_g0), (W_g1, b_g1), (W_g2, b_g2), (W_g3, b_g3),
          (W_g4, b_g4), (W_g5, b_g5), (W_g6, b_g6), (W_g7, b_g7)]
    din_pad = 256
    for i, (w, b) in enumerate(gw):
        dout = _CONV_OUT_PAD[i]
        wp = _pad2(w, din_pad, dout)
        bp = _pad1(b, dout).reshape(1, -1)
        h = _relu_mm(x, wp, bp)
        x = _make_segsum(dout)(h, src_p, dst_p, cptrs[_CHUNK_CFG[dout][0]])
        din_pad = dout

    fin_ws = [
        (_pad2(W_fin0, 16, 16), b_fin0.reshape(1, -1)),
        (W_fin1, b_fin1.reshape(1, -1)),
        (W_fin2, b_fin2.reshape(1, -1)),
        (W_fin3, b_fin3.reshape(1, -1)),
        (_pad2(W_fin4, 8, 8), _pad1(b_fin4, 8).reshape(1, -1)),
    ]
    out = _final(x, fin_ws)
    return out[:N, :1]


# speedup vs baseline: 6.8146x; 1.0627x over previous
import functools

import jax
import jax.numpy as jnp
from jax import lax
from jax.experimental import pallas as pl
from jax.experimental.pallas import tpu as pltpu
from jax.experimental.pallas import tpu_sc as plsc

N = 50000
E = 800000

T = 128
IB = 8
IBT = IB * T
NS = 16
EPAD_EXTRA = 3 * NS * IBT
E_PAD = E + EPAD_EXTRA
OB = 32
DSTPAD = 65536

_CHUNK_CFG = {208: (3584, 14), 160: (6656, 8), 128: (8704, 6),
              112: (12544, 4), 64: (25600, 2), 32: (25088, 2),
              16: (25088, 2)}


def _pad2(w, r, c):
    return jnp.pad(w, ((0, r - w.shape[0]), (0, c - w.shape[1])))


def _pad1(b, c):
    return jnp.pad(b, (0, c - b.shape[0]))


_R = 512


def _enc_body(x_ref, w0, b0, w1, b1, w2, b2, w3, b3, o_ref):
    x = x_ref[...]
    for w, b in ((w0, b0), (w1, b1), (w2, b2), (w3, b3)):
        x = jnp.maximum(
            jnp.dot(x, w[...], preferred_element_type=jnp.float32) + b[...], 0.0)
    o_ref[...] = x


def _encoder(x0, ws):
    nrows, din = x0.shape
    specs = [pl.BlockSpec((_R, din), lambda i: (i, 0))]
    for w, b in ws:
        specs.append(pl.BlockSpec(w.shape, lambda i: (0, 0)))
        specs.append(pl.BlockSpec(b.shape, lambda i: (0, 0)))
    dout = ws[-1][0].shape[1]
    args = []
    for w, b in ws:
        args += [w, b]
    return pl.pallas_call(
        _enc_body,
        grid=(nrows // _R,),
        in_specs=specs,
        out_specs=pl.BlockSpec((_R, dout), lambda i: (i, 0)),
        out_shape=jax.ShapeDtypeStruct((nrows, dout), jnp.float32),
    )(x0, *args)


def _mm_body(x_ref, w_ref, b_ref, o_ref):
    x = jnp.maximum(x_ref[...], 0.0)
    o_ref[...] = (
        jnp.dot(x, w_ref[...], preferred_element_type=jnp.float32) + b_ref[...])


def _relu_mm(x, w, b):
    din, dout = w.shape
    nrows = x.shape[0]
    return pl.pallas_call(
        _mm_body,
        grid=(nrows // _R,),
        in_specs=[
            pl.BlockSpec((_R, din), lambda i: (i, 0)),
            pl.BlockSpec((din, dout), lambda i: (0, 0)),
            pl.BlockSpec((1, dout), lambda i: (0, 0)),
        ],
        out_specs=pl.BlockSpec((_R, dout), lambda i: (i, 0)),
        out_shape=jax.ShapeDtypeStruct((nrows, dout), jnp.float32),
    )(x, w, b)


def _fin_body(x_ref, w0, b0, w1, b1, w2, b2, w3, b3, w4, b4, o_ref):
    x = jnp.maximum(x_ref[...], 0.0)
    for w, b in ((w0, b0), (w1, b1), (w2, b2), (w3, b3)):
        x = jnp.maximum(
            jnp.dot(x, w[...], preferred_element_type=jnp.float32) + b[...], 0.0)
    x = jnp.dot(x, w4[...], preferred_element_type=jnp.float32) + b4[...]
    o_ref[...] = jax.nn.sigmoid(x)


def _final(x, ws):
    nrows, din = x.shape
    specs = [pl.BlockSpec((_R, din), lambda i: (i, 0))]
    args = []
    for w, b in ws:
        specs.append(pl.BlockSpec(w.shape, lambda i: (0, 0)))
        specs.append(pl.BlockSpec(b.shape, lambda i: (0, 0)))
        args += [w, b]
    dout = ws[-1][0].shape[1]
    return pl.pallas_call(
        _fin_body,
        grid=(nrows // _R,),
        in_specs=specs,
        out_specs=pl.BlockSpec((_R, dout), lambda i: (i, 0)),
        out_shape=jax.ShapeDtypeStruct((nrows, dout), jnp.float32),
    )(x, *args)


def _extract(vec, k):
    return jax.lax.squeeze(jax.lax.slice(vec, (k,), (k + 1,)), (0,))


def _make_segsum(d):
    chunk, num_chunks = _CHUNK_CFG[d]
    n_pad = chunk * num_chunks
    rows_per_w = chunk // NS
    nk = rows_per_w // OB
    mesh = plsc.VectorSubcoreMesh(core_axis_name="c", subcore_axis_name="s",
                                  num_cores=2)

    def body(h_hbm, src_hbm, dst_hbm, cptr_hbm, out_hbm,
             acc, srcb, dstb, ldstb, rb, zbuf, cptr_v,
             isem0, isem1, gsem0, gsem1):
        cid = lax.axis_index("c")
        wid = lax.axis_index("s")
        iota = lax.iota(jnp.int32, 16)
        isem = (isem0, isem1)
        gsem = (gsem0, gsem1)

        def zrow(r, carry):
            for j in range(d // 16):
                zbuf[r, pl.ds(j * 16, 16)] = jnp.zeros((16,), jnp.float32)
            return carry
        lax.fori_loop(0, OB, zrow, 0)

        pltpu.sync_copy(cptr_hbm, cptr_v)
        cvec = cptr_v[...]

        def ldst_block(p, base):
            for j in range(IB):
                for q in range(T // 16):
                    v = dstb[p][pl.ds(j * T + q * 16, 16)]
                    inb = (v >= base) & (v < base + chunk)
                    ldstb[p][j, pl.ds(q * 16, 16)] = jnp.where(
                        inb, v - base, chunk + iota)

        for c in range(num_chunks):
            @pl.when(cid == c % 2)
            def _chunk(c=c):
                base = c * chunk
                lo = _extract(cvec, c)
                hi = _extract(cvec, c + 1)
                for k in range(nk):
                    pltpu.sync_copy(
                        zbuf, acc.at[pl.ds(wid * rows_per_w + k * OB, OB)])
                plsc.subcore_barrier()

                lo8 = (lo >> 3) << 3

                def blk_start(n):
                    return pl.multiple_of(lo8 + (n * NS + wid) * IBT, 8)

                nb = (hi - lo8 + (NS * IBT - 1)) >> 14
                nb2 = (nb + 1) >> 1

                s0 = blk_start(0)
                pltpu.async_copy(src_hbm.at[pl.ds(s0, IBT)], srcb[0], isem[0])
                pltpu.async_copy(dst_hbm.at[pl.ds(s0, IBT)], dstb[0], isem[0])

                def block_pair(n2, carry):
                    for p in range(2):
                        n = 2 * n2 + p
                        sn = blk_start(n)
                        pltpu.make_async_copy(
                            src_hbm.at[pl.ds(sn, IBT)], srcb[p],
                            isem[p]).wait()
                        pltpu.make_async_copy(
                            dst_hbm.at[pl.ds(sn, IBT)], dstb[p],
                            isem[p]).wait()
                        sn1 = blk_start(n + 1)
                        pltpu.async_copy(
                            src_hbm.at[pl.ds(sn1, IBT)], srcb[p ^ 1],
                            isem[p ^ 1])
                        pltpu.async_copy(
                            dst_hbm.at[pl.ds(sn1, IBT)], dstb[p ^ 1],
                            isem[p ^ 1])
                        ldst_block(p, base)
                        prev = None
                        for j in range(IB):
                            cp = pltpu.async_copy(
                                h_hbm.at[srcb[p].at[pl.ds(j * T, T)]],
                                rb[j % 2], gsem[j % 2])
                            if prev is not None:
                                prev[0].wait()
                                pltpu.sync_copy(
                                    rb[(j - 1) % 2],
                                    acc.at[ldstb[p].at[j - 1]], add=True)
                            prev = (cp, j)
                        prev[0].wait()
                        pltpu.sync_copy(
                            rb[(IB - 1) % 2],
                            acc.at[ldstb[p].at[IB - 1]], add=True)
                    return carry

                lax.fori_loop(0, nb2, block_pair, 0)
                se = blk_start(2 * nb2)
                pltpu.make_async_copy(
                    src_hbm.at[pl.ds(se, IBT)], srcb[0], isem[0]).wait()
                pltpu.make_async_copy(
                    dst_hbm.at[pl.ds(se, IBT)], dstb[0], isem[0]).wait()
                plsc.subcore_barrier()

                for k in range(nk):
                    r0 = wid * rows_per_w + k * OB
                    pltpu.sync_copy(acc.at[pl.ds(r0, OB)],
                                    out_hbm.at[pl.ds(base + r0, OB)])
                plsc.subcore_barrier()

    return pl.kernel(
        body,
        out_type=jax.ShapeDtypeStruct((n_pad, d), jnp.float32),
        mesh=mesh,
        compiler_params=pltpu.CompilerParams(use_tc_tiling_on_sc=False),
        scratch_types=[
            pltpu.VMEM_SHARED((chunk + 16, d), jnp.float32),
            (pltpu.VMEM((IBT,), jnp.int32), pltpu.VMEM((IBT,), jnp.int32)),
            (pltpu.VMEM((IBT,), jnp.int32), pltpu.VMEM((IBT,), jnp.int32)),
            (pltpu.VMEM((IB, T), jnp.int32), pltpu.VMEM((IB, T), jnp.int32)),
            (pltpu.VMEM((T, d), jnp.float32), pltpu.VMEM((T, d), jnp.float32)),
            pltpu.VMEM((OB, d), jnp.float32),
            pltpu.VMEM((16,), jnp.int32),
            pltpu.SemaphoreType.DMA,
            pltpu.SemaphoreType.DMA,
            pltpu.SemaphoreType.DMA,
            pltpu.SemaphoreType.DMA,
        ],
    )


_CONV_OUT_PAD = [208, 160, 128, 112, 64, 32, 16, 16]


def kernel(one_hot, features, gemme_features, a_res,
           W_enc0, b_enc0, W_enc1, b_enc1, W_enc2, b_enc2, W_enc3, b_enc3,
           W_g0, b_g0, W_g1, b_g1, W_g2, b_g2, W_g3, b_g3,
           W_g4, b_g4, W_g5, b_g5, W_g6, b_g6, W_g7, b_g7,
           W_fin0, b_fin0, W_fin1, b_fin1, W_fin2, b_fin2, W_fin3, b_fin3,
           W_fin4, b_fin4):
    f32 = jnp.float32

    dst_s, src_s = lax.sort((a_res[1], a_res[0]), num_keys=1)
    cptrs = {}
    for chunk, num_chunks in set(_CHUNK_CFG.values()):
        bounds = jnp.arange(1, num_chunks, dtype=jnp.int32) * chunk
        cs = jnp.searchsorted(dst_s, bounds).astype(jnp.int32)
        cptrs[chunk] = jnp.concatenate([
            jnp.zeros((1,), jnp.int32), cs,
            jnp.full((16 - num_chunks,), E, jnp.int32)])
    pad_src = (jnp.arange(EPAD_EXTRA, dtype=jnp.int32) * 977) % N
    pad_dst = jnp.full((EPAD_EXTRA,), DSTPAD, jnp.int32)
    src_p = jnp.concatenate([src_s, pad_src])
    dst_p = jnp.concatenate([dst_s, pad_dst])

    x0 = jnp.pad(jnp.concatenate([one_hot, features], axis=1),
                 ((0, 50176 - N), (0, 12)))
    enc_ws = [
        (_pad2(W_enc0, 160, 32), _pad1(b_enc0, 32).reshape(1, -1)),
        (W_enc1, b_enc1.reshape(1, -1)),
        (W_enc2, b_enc2.reshape(1, -1)),
        (W_enc3, b_enc3.reshape(1, -1)),
    ]
    x = _encoder(x0, enc_ws)

    gw = [(W_g0, b_g0), (W_g1, b_g1), (W_g2, b_g2), (W_g3, b_g3),
          (W_g4, b_g4), (W_g5, b_g5), (W_g6, b_g6), (W_g7, b_g7)]
    din_pad = 256
    for i, (w, b) in enumerate(gw):
        dout = _CONV_OUT_PAD[i]
        wp = _pad2(w, din_pad, dout)
        bp = _pad1(b, dout).reshape(1, -1)
        h = _relu_mm(x, wp, bp)
        x = _make_segsum(dout)(h, src_p, dst_p, cptrs[_CHUNK_CFG[dout][0]])
        din_pad = dout

    fin_ws = [
        (_pad2(W_fin0, 16, 16), b_fin0.reshape(1, -1)),
        (W_fin1, b_fin1.reshape(1, -1)),
        (W_fin2, b_fin2.reshape(1, -1)),
        (W_fin3, b_fin3.reshape(1, -1)),
        (_pad2(W_fin4, 8, 8), _pad1(b_fin4, 8).reshape(1, -1)),
    ]
    out = _final(x, fin_ws)
    return out[:N, :1]
